# baseline (speedup 1.0000x reference)
import functools

import jax
import jax.numpy as jnp
from jax import lax
from jax.experimental import pallas as pl
from jax.experimental.pallas import tpu as pltpu
from jax.experimental.pallas import tpu_sc as plsc

N = 10000
E = 320000
D = 128
DH = D // 2
NC = 2
NS = 16
EC = E // NS
C = 80
CHUNKS = EC // C
NBUF = 5
NGROUPS = CHUNKS // NBUF
EHALF = E // NC
DCHUNKS = EHALF // NS // C
DGROUPS = DCHUNKS // NBUF
RPT = 624
RLAST = N - 15 * RPT
DW = 16
BM = 1000

_mesh = plsc.VectorSubcoreMesh(core_axis_name="c", subcore_axis_name="s")


def _per_tile_rows(s, fn):
    off = s * RPT

    @pl.when(s < NS - 1)
    def _():
        fn(off, RPT)

    @pl.when(s == NS - 1)
    def _():
        fn(off, RLAST)


@functools.partial(
    pl.kernel,
    out_type=jax.ShapeDtypeStruct((NC, N, DW), jnp.float32),
    mesh=_mesh,
    compiler_params=pltpu.CompilerParams(use_tc_tiling_on_sc=False),
    scratch_types=[
        pltpu.VMEM((DCHUNKS, C), jnp.int32),
        pltpu.VMEM((C, DW), jnp.float32),
        pltpu.VMEM_SHARED((N, DW), jnp.float32),
        pltpu.SemaphoreType.DMA,
    ],
)
def _deg_kernel(dst_hbm, ones_hbm, zdeg_hbm, out_hbm, dst_v, ones_v, deg_sh, sem):
    c = lax.axis_index("c")
    s = lax.axis_index("s")
    wid = s * NC + c
    pltpu.sync_copy(dst_hbm.at[wid], dst_v)
    pltpu.sync_copy(ones_hbm, ones_v)
    _per_tile_rows(s, lambda off, n: pltpu.sync_copy(
        zdeg_hbm.at[pl.ds(off, n)], deg_sh.at[pl.ds(off, n)]))
    plsc.subcore_barrier()

    def group(g, carry):
        for b in range(NBUF):
            pltpu.async_copy(ones_v, deg_sh.at[dst_v.at[g * NBUF + b]], sem, add=True)
        for b in range(NBUF):
            pltpu.make_async_copy(ones_hbm, ones_v, sem).wait()
        return carry

    lax.fori_loop(0, DGROUPS, group, 0)
    plsc.subcore_barrier()
    _per_tile_rows(s, lambda off, n: pltpu.sync_copy(
        deg_sh.at[pl.ds(off, n)], out_hbm.at[c, pl.ds(off, n)]))


@functools.partial(
    pl.kernel,
    out_type=jax.ShapeDtypeStruct((NC, N, DH), jnp.float32),
    mesh=_mesh,
    compiler_params=pltpu.CompilerParams(use_tc_tiling_on_sc=False),
    scratch_types=[
        pltpu.VMEM((CHUNKS, C), jnp.int32),
        pltpu.VMEM((CHUNKS, C), jnp.int32),
        pltpu.VMEM((NBUF, C, DH), jnp.float32),
        pltpu.VMEM_SHARED((N, DH), jnp.float32),
        pltpu.SemaphoreType.DMA((2,)),
    ],
)
def _edge_kernel(hd_hbm, src_hbm, dst_hbm, out_hbm,
                 src_v, dst_v, ring, acc_sh, sem):
    c = lax.axis_index("c")
    s = lax.axis_index("s")
    pltpu.sync_copy(src_hbm.at[s], src_v)
    pltpu.sync_copy(dst_hbm.at[s], dst_v)

    hd_c = hd_hbm.at[c]

    def fire_gathers(g):
        for j in range(NBUF):
            pltpu.async_copy(hd_c.at[src_v.at[g * NBUF + j]], ring.at[j],
                             sem.at[0])

    def wait_gathers():
        for j in range(NBUF):
            pltpu.make_async_copy(hd_c.at[pl.ds(0, C)], ring.at[j],
                                  sem.at[0]).wait()

    def fire_scatters(g):
        for j in range(NBUF):
            pltpu.async_copy(ring.at[j], acc_sh.at[dst_v.at[g * NBUF + j]],
                             sem.at[1], add=True)

    def drain_scatters():
        for j in range(NBUF):
            pltpu.make_async_copy(hd_c.at[pl.ds(0, C)], ring.at[j],
                                  sem.at[1]).wait()

    fire_gathers(0)
    _per_tile_rows(s, lambda off, n: pltpu.sync_copy(
        hd_hbm.at[c, pl.ds(off, n)], acc_sh.at[pl.ds(off, n)]))
    plsc.subcore_barrier()

    def group(g, carry):
        wait_gathers()
        fire_scatters(g)
        drain_scatters()

        @pl.when(g + 1 < NGROUPS)
        def _():
            fire_gathers(g + 1)

        return carry

    lax.fori_loop(0, NGROUPS, group, 0)
    plsc.subcore_barrier()
    _per_tile_rows(s, lambda off, n: pltpu.sync_copy(
        acc_sh.at[pl.ds(off, n)], out_hbm.at[c, pl.ds(off, n)]))


def _d_block(degp_ref):
    deg = 1.0 + degp_ref[0, :, 0:1] + degp_ref[1, :, 0:1]
    return lax.rsqrt(deg)


def _split_store(o_ref, res):
    o_ref[0, :, :] = res[:, :DH]
    o_ref[1, :, :] = res[:, DH:]


def _mm1_body(x_ref, w_ref, degp_ref, o_ref):
    d = _d_block(degp_ref)
    _split_store(o_ref, d * jnp.dot(x_ref[...], w_ref[...],
                                    preferred_element_type=jnp.float32))


def _mm2_body(q_ref, degp_ref, b1_ref, w2_ref, o_ref):
    d = _d_block(degp_ref)
    q = jnp.concatenate([q_ref[0], q_ref[1]], axis=-1)
    t = jnp.maximum(d * q + b1_ref[...], 0.0)
    _split_store(o_ref, d * jnp.dot(t, w2_ref[...],
                                    preferred_element_type=jnp.float32))


def _comb_body(r_ref, degp_ref, b2_ref, o_ref):
    d = _d_block(degp_ref)
    r = jnp.concatenate([r_ref[0], r_ref[1]], axis=-1)
    o_ref[...] = d * r + b2_ref[...]


_row_spec = pl.BlockSpec((BM, D), lambda i: (i, 0))
_half_spec = pl.BlockSpec((NC, BM, DH), lambda i: (0, i, 0))
_deg_spec = pl.BlockSpec((NC, BM, DW), lambda i: (0, i, 0))
_w_spec = pl.BlockSpec((D, D), lambda i: (0, 0))
_b_spec = pl.BlockSpec((1, D), lambda i: (0, 0))
_half_out = jax.ShapeDtypeStruct((NC, N, DH), jnp.float32)

_mm1 = pl.pallas_call(
    _mm1_body, grid=(N // BM,),
    in_specs=[_row_spec, _w_spec, _deg_spec],
    out_specs=_half_spec, out_shape=_half_out)

_mm2 = pl.pallas_call(
    _mm2_body, grid=(N // BM,),
    in_specs=[_half_spec, _deg_spec, _b_spec, _w_spec],
    out_specs=_half_spec, out_shape=_half_out)

_comb = pl.pallas_call(
    _comb_body, grid=(N // BM,),
    in_specs=[_half_spec, _deg_spec, _b_spec],
    out_specs=pl.BlockSpec((BM, D), lambda i: (i, 0)),
    out_shape=jax.ShapeDtypeStruct((N, D), jnp.float32))


@jax.jit
def kernel(x, edge_index, W1, b1, W2, b2):
    ei = edge_index.astype(jnp.int32)
    src = ei[0].reshape(NS, CHUNKS, C)
    dst = ei[1].reshape(NS, CHUNKS, C)
    dstd = ei[1].reshape(NC * NS, DCHUNKS, C)
    ones = jnp.ones((C, DW), jnp.float32)
    zdeg = jnp.zeros((N, DW), jnp.float32)
    b1r = b1.reshape(1, D)
    b2r = b2.reshape(1, D)

    degp = _deg_kernel(dstd, ones, zdeg)
    hd1 = _mm1(x, W1, degp)
    q = _edge_kernel(hd1, src, dst)
    hd2 = _mm2(q, degp, b1r, W2)
    r = _edge_kernel(hd2, src, dst)
    return _comb(r, degp, b2r)


# speedup vs baseline: 1.1538x; 1.1538x over previous
import functools

import jax
import jax.numpy as jnp
from jax import lax
from jax.experimental import pallas as pl
from jax.experimental.pallas import tpu as pltpu
from jax.experimental.pallas import tpu_sc as plsc

N = 10000
E = 320000
D = 128
DH = D // 2
NC = 2
NS = 16
EC = E // NS
C = 80
CHUNKS = EC // C
NBUF = 5
NGROUPS = CHUNKS // NBUF
EHALF = E // NC
DCHUNKS = EHALF // NS // C
DGROUPS = DCHUNKS // NBUF
RPT = 624
RLAST = N - 15 * RPT
DW = 16
BM = 1000

_mesh = plsc.VectorSubcoreMesh(core_axis_name="c", subcore_axis_name="s")


def _per_tile_rows(s, fn):
    off = s * RPT

    @pl.when(s < NS - 1)
    def _():
        fn(off, RPT)

    @pl.when(s == NS - 1)
    def _():
        fn(off, RLAST)


@functools.partial(
    pl.kernel,
    out_type=jax.ShapeDtypeStruct((NC, N, DW), jnp.float32),
    mesh=_mesh,
    compiler_params=pltpu.CompilerParams(use_tc_tiling_on_sc=False),
    scratch_types=[
        pltpu.VMEM((DCHUNKS, C), jnp.int32),
        pltpu.VMEM((C, DW), jnp.float32),
        pltpu.VMEM_SHARED((N, DW), jnp.float32),
        pltpu.SemaphoreType.DMA,
    ],
)
def _deg_kernel(dst_hbm, ones_hbm, zdeg_hbm, out_hbm, dst_v, ones_v, deg_sh, sem):
    c = lax.axis_index("c")
    s = lax.axis_index("s")
    wid = s * NC + c
    pltpu.sync_copy(dst_hbm.at[wid], dst_v)
    pltpu.sync_copy(ones_hbm, ones_v)
    _per_tile_rows(s, lambda off, n: pltpu.sync_copy(
        zdeg_hbm.at[pl.ds(off, n)], deg_sh.at[pl.ds(off, n)]))
    plsc.subcore_barrier()

    def group(g, carry):
        for b in range(NBUF):
            pltpu.async_copy(ones_v, deg_sh.at[dst_v.at[g * NBUF + b]], sem, add=True)
        for b in range(NBUF):
            pltpu.make_async_copy(ones_hbm, ones_v, sem).wait()
        return carry

    lax.fori_loop(0, DGROUPS, group, 0)
    plsc.subcore_barrier()
    _per_tile_rows(s, lambda off, n: pltpu.sync_copy(
        deg_sh.at[pl.ds(off, n)], out_hbm.at[c, pl.ds(off, n)]))


@functools.partial(
    pl.kernel,
    out_type=jax.ShapeDtypeStruct((NC, N, DH), jnp.float32),
    mesh=_mesh,
    compiler_params=pltpu.CompilerParams(use_tc_tiling_on_sc=False),
    scratch_types=[
        pltpu.VMEM((CHUNKS, C), jnp.int32),
        pltpu.VMEM((CHUNKS, C), jnp.int32),
        pltpu.VMEM((NBUF, C, DH), jnp.float32),
        pltpu.VMEM_SHARED((N, DH), jnp.float32),
        pltpu.SemaphoreType.DMA((2,)),
    ],
)
def _edge_kernel(hd_hbm, src_hbm, dst_hbm, out_hbm,
                 src_v, dst_v, ring, acc_sh, sem):
    c = lax.axis_index("c")
    s = lax.axis_index("s")
    pltpu.sync_copy(src_hbm.at[s], src_v)
    pltpu.sync_copy(dst_hbm.at[s], dst_v)

    hd_c = hd_hbm.at[c]

    def fire_gathers(g):
        for j in range(NBUF):
            pltpu.async_copy(hd_c.at[src_v.at[g * NBUF + j]], ring.at[j],
                             sem.at[0])

    def drain_scatters():
        for j in range(NBUF):
            pltpu.make_async_copy(hd_c.at[pl.ds(0, C)], ring.at[j],
                                  sem.at[1]).wait()

    fire_gathers(0)
    _per_tile_rows(s, lambda off, n: pltpu.sync_copy(
        hd_hbm.at[c, pl.ds(off, n)], acc_sh.at[pl.ds(off, n)]))
    plsc.subcore_barrier()

    def group(g, carry):
        for b in range(NBUF):
            pltpu.make_async_copy(hd_c.at[pl.ds(0, C)], ring.at[b],
                                  sem.at[0]).wait()
            pltpu.async_copy(ring.at[b], acc_sh.at[dst_v.at[g * NBUF + b]],
                             sem.at[1], add=True)
        drain_scatters()

        @pl.when(g + 1 < NGROUPS)
        def _():
            fire_gathers(g + 1)

        return carry

    lax.fori_loop(0, NGROUPS, group, 0)
    plsc.subcore_barrier()
    _per_tile_rows(s, lambda off, n: pltpu.sync_copy(
        acc_sh.at[pl.ds(off, n)], out_hbm.at[c, pl.ds(off, n)]))


def _d_block(degp_ref):
    deg = 1.0 + degp_ref[0, :, 0:1] + degp_ref[1, :, 0:1]
    return lax.rsqrt(deg)


def _split_store(o_ref, res):
    o_ref[0, :, :] = res[:, :DH]
    o_ref[1, :, :] = res[:, DH:]


def _mm1_body(x_ref, w_ref, degp_ref, o_ref):
    d = _d_block(degp_ref)
    _split_store(o_ref, d * jnp.dot(x_ref[...], w_ref[...],
                                    preferred_element_type=jnp.float32))


def _mm2_body(q_ref, degp_ref, b1_ref, w2_ref, o_ref):
    d = _d_block(degp_ref)
    q = jnp.concatenate([q_ref[0], q_ref[1]], axis=-1)
    t = jnp.maximum(d * q + b1_ref[...], 0.0)
    _split_store(o_ref, d * jnp.dot(t, w2_ref[...],
                                    preferred_element_type=jnp.float32))


def _comb_body(r_ref, degp_ref, b2_ref, o_ref):
    d = _d_block(degp_ref)
    r = jnp.concatenate([r_ref[0], r_ref[1]], axis=-1)
    o_ref[...] = d * r + b2_ref[...]


_row_spec = pl.BlockSpec((BM, D), lambda i: (i, 0))
_half_spec = pl.BlockSpec((NC, BM, DH), lambda i: (0, i, 0))
_deg_spec = pl.BlockSpec((NC, BM, DW), lambda i: (0, i, 0))
_w_spec = pl.BlockSpec((D, D), lambda i: (0, 0))
_b_spec = pl.BlockSpec((1, D), lambda i: (0, 0))
_half_out = jax.ShapeDtypeStruct((NC, N, DH), jnp.float32)

_mm1 = pl.pallas_call(
    _mm1_body, grid=(N // BM,),
    in_specs=[_row_spec, _w_spec, _deg_spec],
    out_specs=_half_spec, out_shape=_half_out)

_mm2 = pl.pallas_call(
    _mm2_body, grid=(N // BM,),
    in_specs=[_half_spec, _deg_spec, _b_spec, _w_spec],
    out_specs=_half_spec, out_shape=_half_out)

_comb = pl.pallas_call(
    _comb_body, grid=(N // BM,),
    in_specs=[_half_spec, _deg_spec, _b_spec],
    out_specs=pl.BlockSpec((BM, D), lambda i: (i, 0)),
    out_shape=jax.ShapeDtypeStruct((N, D), jnp.float32))


@jax.jit
def kernel(x, edge_index, W1, b1, W2, b2):
    ei = edge_index.astype(jnp.int32)
    src = ei[0].reshape(NS, CHUNKS, C)
    dst = ei[1].reshape(NS, CHUNKS, C)
    dstd = ei[1].reshape(NC * NS, DCHUNKS, C)
    ones = jnp.ones((C, DW), jnp.float32)
    zdeg = jnp.zeros((N, DW), jnp.float32)
    b1r = b1.reshape(1, D)
    b2r = b2.reshape(1, D)

    degp = _deg_kernel(dstd, ones, zdeg)
    hd1 = _mm1(x, W1, degp)
    q = _edge_kernel(hd1, src, dst)
    hd2 = _mm2(q, degp, b1r, W2)
    r = _edge_kernel(hd2, src, dst)
    return _comb(r, degp, b2r)


# speedup vs baseline: 1.1723x; 1.0160x over previous
import functools

import jax
import jax.numpy as jnp
from jax import lax
from jax.experimental import pallas as pl
from jax.experimental.pallas import tpu as pltpu
from jax.experimental.pallas import tpu_sc as plsc

N = 10000
E = 320000
D = 128
DH = D // 2
NC = 2
NS = 16
EC = E // NS
C = 80
CHUNKS = EC // C
NBUF = 5
NGROUPS = CHUNKS // NBUF
EHALF = E // NC
DCHUNKS = EHALF // NS // C
DGROUPS = DCHUNKS // NBUF
RPT = 624
RLAST = N - 15 * RPT
DW = 16
BM = 2000

_mesh = plsc.VectorSubcoreMesh(core_axis_name="c", subcore_axis_name="s")


def _per_tile_rows(s, fn):
    off = s * RPT

    @pl.when(s < NS - 1)
    def _():
        fn(off, RPT)

    @pl.when(s == NS - 1)
    def _():
        fn(off, RLAST)


@functools.partial(
    pl.kernel,
    out_type=jax.ShapeDtypeStruct((NC, N, DW), jnp.float32),
    mesh=_mesh,
    compiler_params=pltpu.CompilerParams(use_tc_tiling_on_sc=False),
    scratch_types=[
        pltpu.VMEM((DCHUNKS, C), jnp.int32),
        pltpu.VMEM((C, DW), jnp.float32),
        pltpu.VMEM_SHARED((N, DW), jnp.float32),
        pltpu.SemaphoreType.DMA,
    ],
)
def _deg_kernel(dst_hbm, ones_hbm, zdeg_hbm, out_hbm, dst_v, ones_v, deg_sh, sem):
    c = lax.axis_index("c")
    s = lax.axis_index("s")
    wid = s * NC + c
    pltpu.sync_copy(dst_hbm.at[wid], dst_v)
    pltpu.sync_copy(ones_hbm, ones_v)
    _per_tile_rows(s, lambda off, n: pltpu.sync_copy(
        zdeg_hbm.at[pl.ds(off, n)], deg_sh.at[pl.ds(off, n)]))
    plsc.subcore_barrier()

    def group(g, carry):
        for b in range(NBUF):
            pltpu.async_copy(ones_v, deg_sh.at[dst_v.at[g * NBUF + b]], sem, add=True)
        for b in range(NBUF):
            pltpu.make_async_copy(ones_hbm, ones_v, sem).wait()
        return carry

    lax.fori_loop(0, DGROUPS, group, 0)
    plsc.subcore_barrier()
    _per_tile_rows(s, lambda off, n: pltpu.sync_copy(
        deg_sh.at[pl.ds(off, n)], out_hbm.at[c, pl.ds(off, n)]))


@functools.partial(
    pl.kernel,
    out_type=jax.ShapeDtypeStruct((NC, N, DH), jnp.float32),
    mesh=_mesh,
    compiler_params=pltpu.CompilerParams(use_tc_tiling_on_sc=False),
    scratch_types=[
        pltpu.VMEM((CHUNKS, C), jnp.int32),
        pltpu.VMEM((CHUNKS, C), jnp.int32),
        pltpu.VMEM((NBUF, C, DH), jnp.float32),
        pltpu.VMEM_SHARED((N, DH), jnp.float32),
        pltpu.SemaphoreType.DMA((2,)),
    ],
)
def _edge_kernel(hd_hbm, src_hbm, dst_hbm, out_hbm,
                 src_v, dst_v, ring, acc_sh, sem):
    c = lax.axis_index("c")
    s = lax.axis_index("s")
    pltpu.sync_copy(src_hbm.at[s], src_v)
    pltpu.sync_copy(dst_hbm.at[s], dst_v)

    hd_c = hd_hbm.at[c]

    def fire_gathers(g):
        for j in range(NBUF):
            pltpu.async_copy(hd_c.at[src_v.at[g * NBUF + j]], ring.at[j],
                             sem.at[0])

    def drain_scatters():
        for j in range(NBUF):
            pltpu.make_async_copy(hd_c.at[pl.ds(0, C)], ring.at[j],
                                  sem.at[1]).wait()

    fire_gathers(0)
    _per_tile_rows(s, lambda off, n: pltpu.sync_copy(
        hd_hbm.at[c, pl.ds(off, n)], acc_sh.at[pl.ds(off, n)]))
    plsc.subcore_barrier()

    def group(g, carry):
        for b in range(NBUF):
            pltpu.make_async_copy(hd_c.at[pl.ds(0, C)], ring.at[b],
                                  sem.at[0]).wait()
            pltpu.async_copy(ring.at[b], acc_sh.at[dst_v.at[g * NBUF + b]],
                             sem.at[1], add=True)
        drain_scatters()

        @pl.when(g + 1 < NGROUPS)
        def _():
            fire_gathers(g + 1)

        return carry

    lax.fori_loop(0, NGROUPS, group, 0)
    plsc.subcore_barrier()
    _per_tile_rows(s, lambda off, n: pltpu.sync_copy(
        acc_sh.at[pl.ds(off, n)], out_hbm.at[c, pl.ds(off, n)]))


def _d_block(degp_ref):
    deg = 1.0 + degp_ref[0, :, 0:1] + degp_ref[1, :, 0:1]
    return lax.rsqrt(deg)


def _split_store(o_ref, res):
    o_ref[0, :, :] = res[:, :DH]
    o_ref[1, :, :] = res[:, DH:]


def _mm1_body(x_ref, w_ref, degp_ref, o_ref):
    d = _d_block(degp_ref)
    _split_store(o_ref, d * jnp.dot(x_ref[...], w_ref[...],
                                    preferred_element_type=jnp.float32))


def _mm2_body(q_ref, degp_ref, b1_ref, w2_ref, o_ref):
    d = _d_block(degp_ref)
    q = jnp.concatenate([q_ref[0], q_ref[1]], axis=-1)
    t = jnp.maximum(d * q + b1_ref[...], 0.0)
    _split_store(o_ref, d * jnp.dot(t, w2_ref[...],
                                    preferred_element_type=jnp.float32))


def _comb_body(r_ref, degp_ref, b2_ref, o_ref):
    d = _d_block(degp_ref)
    r = jnp.concatenate([r_ref[0], r_ref[1]], axis=-1)
    o_ref[...] = d * r + b2_ref[...]


_row_spec = pl.BlockSpec((BM, D), lambda i: (i, 0))
_half_spec = pl.BlockSpec((NC, BM, DH), lambda i: (0, i, 0))
_deg_spec = pl.BlockSpec((NC, BM, DW), lambda i: (0, i, 0))
_w_spec = pl.BlockSpec((D, D), lambda i: (0, 0))
_b_spec = pl.BlockSpec((1, D), lambda i: (0, 0))
_half_out = jax.ShapeDtypeStruct((NC, N, DH), jnp.float32)

_mm1 = pl.pallas_call(
    _mm1_body, grid=(N // BM,),
    in_specs=[_row_spec, _w_spec, _deg_spec],
    out_specs=_half_spec, out_shape=_half_out)

_mm2 = pl.pallas_call(
    _mm2_body, grid=(N // BM,),
    in_specs=[_half_spec, _deg_spec, _b_spec, _w_spec],
    out_specs=_half_spec, out_shape=_half_out)

_comb = pl.pallas_call(
    _comb_body, grid=(N // BM,),
    in_specs=[_half_spec, _deg_spec, _b_spec],
    out_specs=pl.BlockSpec((BM, D), lambda i: (i, 0)),
    out_shape=jax.ShapeDtypeStruct((N, D), jnp.float32))


@jax.jit
def kernel(x, edge_index, W1, b1, W2, b2):
    ei = edge_index.astype(jnp.int32)
    src = ei[0].reshape(NS, CHUNKS, C)
    dst = ei[1].reshape(NS, CHUNKS, C)
    dstd = ei[1].reshape(NC * NS, DCHUNKS, C)
    ones = jnp.ones((C, DW), jnp.float32)
    zdeg = jnp.zeros((N, DW), jnp.float32)
    b1r = b1.reshape(1, D)
    b2r = b2.reshape(1, D)

    degp = _deg_kernel(dstd, ones, zdeg)
    hd1 = _mm1(x, W1, degp)
    q = _edge_kernel(hd1, src, dst)
    hd2 = _mm2(q, degp, b1r, W2)
    r = _edge_kernel(hd2, src, dst)
    return _comb(r, degp, b2r)


# speedup vs baseline: 1.1875x; 1.0130x over previous
import functools

import jax
import jax.numpy as jnp
from jax import lax
from jax.experimental import pallas as pl
from jax.experimental.pallas import tpu as pltpu
from jax.experimental.pallas import tpu_sc as plsc

N = 10000
E = 320000
D = 128
DH = D // 2
NC = 2
NS = 16
EC = E // NS
C = 80
CHUNKS = EC // C
NBUF = 5
NGROUPS = CHUNKS // NBUF
EHALF = E // NC
DCHUNKS = EHALF // NS // C
DGROUPS = DCHUNKS // NBUF
RPT = 624
RLAST = N - 15 * RPT
DW = 16
BM = 5000

_mesh = plsc.VectorSubcoreMesh(core_axis_name="c", subcore_axis_name="s")


def _per_tile_rows(s, fn):
    off = s * RPT

    @pl.when(s < NS - 1)
    def _():
        fn(off, RPT)

    @pl.when(s == NS - 1)
    def _():
        fn(off, RLAST)


@functools.partial(
    pl.kernel,
    out_type=jax.ShapeDtypeStruct((NC, N, DW), jnp.float32),
    mesh=_mesh,
    compiler_params=pltpu.CompilerParams(use_tc_tiling_on_sc=False),
    scratch_types=[
        pltpu.VMEM((DCHUNKS, C), jnp.int32),
        pltpu.VMEM((C, DW), jnp.float32),
        pltpu.VMEM_SHARED((N, DW), jnp.float32),
        pltpu.SemaphoreType.DMA,
    ],
)
def _deg_kernel(dst_hbm, ones_hbm, zdeg_hbm, out_hbm, dst_v, ones_v, deg_sh, sem):
    c = lax.axis_index("c")
    s = lax.axis_index("s")
    wid = s * NC + c
    pltpu.sync_copy(dst_hbm.at[wid], dst_v)
    pltpu.sync_copy(ones_hbm, ones_v)
    _per_tile_rows(s, lambda off, n: pltpu.sync_copy(
        zdeg_hbm.at[pl.ds(off, n)], deg_sh.at[pl.ds(off, n)]))
    plsc.subcore_barrier()

    def group(g, carry):
        for b in range(NBUF):
            pltpu.async_copy(ones_v, deg_sh.at[dst_v.at[g * NBUF + b]], sem, add=True)
        for b in range(NBUF):
            pltpu.make_async_copy(ones_hbm, ones_v, sem).wait()
        return carry

    lax.fori_loop(0, DGROUPS, group, 0)
    plsc.subcore_barrier()
    _per_tile_rows(s, lambda off, n: pltpu.sync_copy(
        deg_sh.at[pl.ds(off, n)], out_hbm.at[c, pl.ds(off, n)]))


@functools.partial(
    pl.kernel,
    out_type=jax.ShapeDtypeStruct((NC, N, DH), jnp.float32),
    mesh=_mesh,
    compiler_params=pltpu.CompilerParams(use_tc_tiling_on_sc=False),
    scratch_types=[
        pltpu.VMEM((CHUNKS, C), jnp.int32),
        pltpu.VMEM((CHUNKS, C), jnp.int32),
        pltpu.VMEM((NBUF, C, DH), jnp.float32),
        pltpu.VMEM_SHARED((N, DH), jnp.float32),
        pltpu.SemaphoreType.DMA((2,)),
    ],
)
def _edge_kernel(hd_hbm, src_hbm, dst_hbm, out_hbm,
                 src_v, dst_v, ring, acc_sh, sem):
    c = lax.axis_index("c")
    s = lax.axis_index("s")
    pltpu.sync_copy(src_hbm.at[s], src_v)
    pltpu.sync_copy(dst_hbm.at[s], dst_v)

    hd_c = hd_hbm.at[c]

    def fire_gathers(g):
        for j in range(NBUF):
            pltpu.async_copy(hd_c.at[src_v.at[g * NBUF + j]], ring.at[j],
                             sem.at[0])

    def drain_scatters():
        for j in range(NBUF):
            pltpu.make_async_copy(hd_c.at[pl.ds(0, C)], ring.at[j],
                                  sem.at[1]).wait()

    fire_gathers(0)
    _per_tile_rows(s, lambda off, n: pltpu.sync_copy(
        hd_hbm.at[c, pl.ds(off, n)], acc_sh.at[pl.ds(off, n)]))
    plsc.subcore_barrier()

    def group(g, carry):
        for b in range(NBUF):
            pltpu.make_async_copy(hd_c.at[pl.ds(0, C)], ring.at[b],
                                  sem.at[0]).wait()
            pltpu.async_copy(ring.at[b], acc_sh.at[dst_v.at[g * NBUF + b]],
                             sem.at[1], add=True)
        drain_scatters()

        @pl.when(g + 1 < NGROUPS)
        def _():
            fire_gathers(g + 1)

        return carry

    lax.fori_loop(0, NGROUPS, group, 0)
    plsc.subcore_barrier()
    _per_tile_rows(s, lambda off, n: pltpu.sync_copy(
        acc_sh.at[pl.ds(off, n)], out_hbm.at[c, pl.ds(off, n)]))


def _d_block(degp_ref):
    deg = 1.0 + degp_ref[0, :, 0:1] + degp_ref[1, :, 0:1]
    return lax.rsqrt(deg)


def _split_store(o_ref, res):
    o_ref[0, :, :] = res[:, :DH]
    o_ref[1, :, :] = res[:, DH:]


def _mm1_body(x_ref, w_ref, degp_ref, o_ref):
    d = _d_block(degp_ref)
    _split_store(o_ref, d * jnp.dot(x_ref[...], w_ref[...],
                                    preferred_element_type=jnp.float32))


def _mm2_body(q_ref, degp_ref, b1_ref, w2_ref, o_ref):
    d = _d_block(degp_ref)
    q = jnp.concatenate([q_ref[0], q_ref[1]], axis=-1)
    t = jnp.maximum(d * q + b1_ref[...], 0.0)
    _split_store(o_ref, d * jnp.dot(t, w2_ref[...],
                                    preferred_element_type=jnp.float32))


def _comb_body(r_ref, degp_ref, b2_ref, o_ref):
    d = _d_block(degp_ref)
    r = jnp.concatenate([r_ref[0], r_ref[1]], axis=-1)
    o_ref[...] = d * r + b2_ref[...]


_row_spec = pl.BlockSpec((BM, D), lambda i: (i, 0))
_half_spec = pl.BlockSpec((NC, BM, DH), lambda i: (0, i, 0))
_deg_spec = pl.BlockSpec((NC, BM, DW), lambda i: (0, i, 0))
_w_spec = pl.BlockSpec((D, D), lambda i: (0, 0))
_b_spec = pl.BlockSpec((1, D), lambda i: (0, 0))
_half_out = jax.ShapeDtypeStruct((NC, N, DH), jnp.float32)

_mm1 = pl.pallas_call(
    _mm1_body, grid=(N // BM,),
    in_specs=[_row_spec, _w_spec, _deg_spec],
    out_specs=_half_spec, out_shape=_half_out)

_mm2 = pl.pallas_call(
    _mm2_body, grid=(N // BM,),
    in_specs=[_half_spec, _deg_spec, _b_spec, _w_spec],
    out_specs=_half_spec, out_shape=_half_out)

_comb = pl.pallas_call(
    _comb_body, grid=(N // BM,),
    in_specs=[_half_spec, _deg_spec, _b_spec],
    out_specs=pl.BlockSpec((BM, D), lambda i: (i, 0)),
    out_shape=jax.ShapeDtypeStruct((N, D), jnp.float32))


@jax.jit
def kernel(x, edge_index, W1, b1, W2, b2):
    ei = edge_index.astype(jnp.int32)
    src = ei[0].reshape(NS, CHUNKS, C)
    dst = ei[1].reshape(NS, CHUNKS, C)
    dstd = ei[1].reshape(NC * NS, DCHUNKS, C)
    ones = jnp.ones((C, DW), jnp.float32)
    zdeg = jnp.zeros((N, DW), jnp.float32)
    b1r = b1.reshape(1, D)
    b2r = b2.reshape(1, D)

    degp = _deg_kernel(dstd, ones, zdeg)
    hd1 = _mm1(x, W1, degp)
    q = _edge_kernel(hd1, src, dst)
    hd2 = _mm2(q, degp, b1r, W2)
    r = _edge_kernel(hd2, src, dst)
    return _comb(r, degp, b2r)


# speedup vs baseline: 1.2774x; 1.0757x over previous
import functools

import jax
import jax.numpy as jnp
from jax import lax
from jax.experimental import pallas as pl
from jax.experimental.pallas import tpu as pltpu
from jax.experimental.pallas import tpu_sc as plsc

N = 10000
E = 320000
D = 128
DH = D // 2
NC = 2
NS = 16
EC = E // NS
C = 80
CHUNKS = EC // C
NBUF = 5
NGROUPS = CHUNKS // NBUF
EHALF = E // NC
DCHUNKS = EHALF // NS // C
DGROUPS = DCHUNKS // NBUF
RPT = 624
RLAST = N - 15 * RPT
DW = 16
BM = 5000

_mesh = plsc.VectorSubcoreMesh(core_axis_name="c", subcore_axis_name="s")


def _per_tile_rows(s, fn):
    off = s * RPT

    @pl.when(s < NS - 1)
    def _():
        fn(off, RPT)

    @pl.when(s == NS - 1)
    def _():
        fn(off, RLAST)


@functools.partial(
    pl.kernel,
    out_type=jax.ShapeDtypeStruct((NC, N, DW), jnp.float32),
    mesh=_mesh,
    compiler_params=pltpu.CompilerParams(use_tc_tiling_on_sc=False),
    scratch_types=[
        pltpu.VMEM((DCHUNKS, C), jnp.int32),
        pltpu.VMEM((C, DW), jnp.float32),
        pltpu.VMEM_SHARED((N, DW), jnp.float32),
        pltpu.SemaphoreType.DMA,
    ],
)
def _deg_kernel(dst_hbm, ones_hbm, zdeg_hbm, out_hbm, dst_v, ones_v, deg_sh, sem):
    c = lax.axis_index("c")
    s = lax.axis_index("s")
    wid = s * NC + c
    pltpu.sync_copy(dst_hbm.at[wid], dst_v)
    pltpu.sync_copy(ones_hbm, ones_v)
    _per_tile_rows(s, lambda off, n: pltpu.sync_copy(
        zdeg_hbm.at[pl.ds(off, n)], deg_sh.at[pl.ds(off, n)]))
    plsc.subcore_barrier()

    def group(g, carry):
        for b in range(NBUF):
            pltpu.async_copy(ones_v, deg_sh.at[dst_v.at[g * NBUF + b]], sem, add=True)
        for b in range(NBUF):
            pltpu.make_async_copy(ones_hbm, ones_v, sem).wait()
        return carry

    lax.fori_loop(0, DGROUPS, group, 0)
    plsc.subcore_barrier()
    _per_tile_rows(s, lambda off, n: pltpu.sync_copy(
        deg_sh.at[pl.ds(off, n)], out_hbm.at[c, pl.ds(off, n)]))


@functools.partial(
    pl.kernel,
    out_type=jax.ShapeDtypeStruct((NC, N, DH), jnp.float32),
    mesh=_mesh,
    compiler_params=pltpu.CompilerParams(use_tc_tiling_on_sc=False),
    scratch_types=[
        pltpu.VMEM((CHUNKS, C), jnp.int32),
        pltpu.VMEM((CHUNKS, C), jnp.int32),
        pltpu.VMEM((NBUF, C, DH), jnp.float32),
        pltpu.VMEM_SHARED((N, DH), jnp.float32),
        pltpu.SemaphoreType.DMA((2,)),
    ],
)
def _edge_kernel(hd_hbm, src_hbm, dst_hbm, out_hbm,
                 src_v, dst_v, ring, acc_sh, sem):
    c = lax.axis_index("c")
    s = lax.axis_index("s")
    pltpu.sync_copy(src_hbm.at[s], src_v)
    pltpu.sync_copy(dst_hbm.at[s], dst_v)

    hd_c = hd_hbm.at[c]

    def fire_gathers(g):
        for j in range(NBUF):
            pltpu.async_copy(hd_c.at[src_v.at[g * NBUF + j]], ring.at[j],
                             sem.at[0])

    def drain_scatters():
        for j in range(NBUF):
            pltpu.make_async_copy(hd_c.at[pl.ds(0, C)], ring.at[j],
                                  sem.at[1]).wait()

    fire_gathers(0)
    _per_tile_rows(s, lambda off, n: pltpu.sync_copy(
        hd_hbm.at[c, pl.ds(off, n)], acc_sh.at[pl.ds(off, n)]))
    plsc.subcore_barrier()

    def group(g, carry):
        for b in range(NBUF):
            pltpu.make_async_copy(hd_c.at[pl.ds(0, C)], ring.at[b],
                                  sem.at[0]).wait()
            pltpu.async_copy(ring.at[b], acc_sh.at[dst_v.at[g * NBUF + b]],
                             sem.at[1], add=True)

        @pl.when(g + 1 < NGROUPS)
        def _():
            for b in range(NBUF):
                pltpu.make_async_copy(hd_c.at[pl.ds(0, C)], ring.at[b],
                                      sem.at[1]).wait()
                pltpu.async_copy(hd_c.at[src_v.at[(g + 1) * NBUF + b]],
                                 ring.at[b], sem.at[0])

        @pl.when(g + 1 == NGROUPS)
        def _():
            drain_scatters()

        return carry

    lax.fori_loop(0, NGROUPS, group, 0)
    plsc.subcore_barrier()
    _per_tile_rows(s, lambda off, n: pltpu.sync_copy(
        acc_sh.at[pl.ds(off, n)], out_hbm.at[c, pl.ds(off, n)]))


def _d_block(degp_ref):
    deg = 1.0 + degp_ref[0, :, 0:1] + degp_ref[1, :, 0:1]
    return lax.rsqrt(deg)


def _split_store(o_ref, res):
    o_ref[0, :, :] = res[:, :DH]
    o_ref[1, :, :] = res[:, DH:]


def _mm1_body(x_ref, w_ref, degp_ref, o_ref):
    d = _d_block(degp_ref)
    _split_store(o_ref, d * jnp.dot(x_ref[...], w_ref[...],
                                    preferred_element_type=jnp.float32))


def _mm2_body(q_ref, degp_ref, b1_ref, w2_ref, o_ref):
    d = _d_block(degp_ref)
    q = jnp.concatenate([q_ref[0], q_ref[1]], axis=-1)
    t = jnp.maximum(d * q + b1_ref[...], 0.0)
    _split_store(o_ref, d * jnp.dot(t, w2_ref[...],
                                    preferred_element_type=jnp.float32))


def _comb_body(r_ref, degp_ref, b2_ref, o_ref):
    d = _d_block(degp_ref)
    r = jnp.concatenate([r_ref[0], r_ref[1]], axis=-1)
    o_ref[...] = d * r + b2_ref[...]


_row_spec = pl.BlockSpec((BM, D), lambda i: (i, 0))
_half_spec = pl.BlockSpec((NC, BM, DH), lambda i: (0, i, 0))
_deg_spec = pl.BlockSpec((NC, BM, DW), lambda i: (0, i, 0))
_w_spec = pl.BlockSpec((D, D), lambda i: (0, 0))
_b_spec = pl.BlockSpec((1, D), lambda i: (0, 0))
_half_out = jax.ShapeDtypeStruct((NC, N, DH), jnp.float32)

_mm1 = pl.pallas_call(
    _mm1_body, grid=(N // BM,),
    in_specs=[_row_spec, _w_spec, _deg_spec],
    out_specs=_half_spec, out_shape=_half_out)

_mm2 = pl.pallas_call(
    _mm2_body, grid=(N // BM,),
    in_specs=[_half_spec, _deg_spec, _b_spec, _w_spec],
    out_specs=_half_spec, out_shape=_half_out)

_comb = pl.pallas_call(
    _comb_body, grid=(N // BM,),
    in_specs=[_half_spec, _deg_spec, _b_spec],
    out_specs=pl.BlockSpec((BM, D), lambda i: (i, 0)),
    out_shape=jax.ShapeDtypeStruct((N, D), jnp.float32))


@jax.jit
def kernel(x, edge_index, W1, b1, W2, b2):
    ei = edge_index.astype(jnp.int32)
    src = ei[0].reshape(NS, CHUNKS, C)
    dst = ei[1].reshape(NS, CHUNKS, C)
    dstd = ei[1].reshape(NC * NS, DCHUNKS, C)
    ones = jnp.ones((C, DW), jnp.float32)
    zdeg = jnp.zeros((N, DW), jnp.float32)
    b1r = b1.reshape(1, D)
    b2r = b2.reshape(1, D)

    degp = _deg_kernel(dstd, ones, zdeg)
    hd1 = _mm1(x, W1, degp)
    q = _edge_kernel(hd1, src, dst)
    hd2 = _mm2(q, degp, b1r, W2)
    r = _edge_kernel(hd2, src, dst)
    return _comb(r, degp, b2r)


# speedup vs baseline: 1.2795x; 1.0016x over previous
import functools

import jax
import jax.numpy as jnp
from jax import lax
from jax.experimental import pallas as pl
from jax.experimental.pallas import tpu as pltpu
from jax.experimental.pallas import tpu_sc as plsc

N = 10000
E = 320000
D = 128
DH = D // 2
NC = 2
NS = 16
EC = E // NS
C = 80
CHUNKS = EC // C
NBUF = 5
NGROUPS = CHUNKS // NBUF
EHALF = E // NC
DCHUNKS = EHALF // NS // C
DGROUPS = DCHUNKS // NBUF
RPT = 624
RLAST = N - 15 * RPT
DW = 16
BM = 5000

_mesh = plsc.VectorSubcoreMesh(core_axis_name="c", subcore_axis_name="s")


def _per_tile_rows(s, fn):
    off = s * RPT

    @pl.when(s < NS - 1)
    def _():
        fn(off, RPT)

    @pl.when(s == NS - 1)
    def _():
        fn(off, RLAST)


@functools.partial(
    pl.kernel,
    out_type=jax.ShapeDtypeStruct((NC, N, DW), jnp.float32),
    mesh=_mesh,
    compiler_params=pltpu.CompilerParams(use_tc_tiling_on_sc=False),
    scratch_types=[
        pltpu.VMEM((DCHUNKS, C), jnp.int32),
        pltpu.VMEM((C, DW), jnp.float32),
        pltpu.VMEM_SHARED((N, DW), jnp.float32),
        pltpu.SemaphoreType.DMA,
    ],
)
def _deg_kernel(dst_hbm, ones_hbm, zdeg_hbm, out_hbm, dst_v, ones_v, deg_sh, sem):
    c = lax.axis_index("c")
    s = lax.axis_index("s")
    wid = s * NC + c
    pltpu.sync_copy(dst_hbm.at[wid], dst_v)
    pltpu.sync_copy(ones_hbm, ones_v)
    _per_tile_rows(s, lambda off, n: pltpu.sync_copy(
        zdeg_hbm.at[pl.ds(off, n)], deg_sh.at[pl.ds(off, n)]))
    plsc.subcore_barrier()

    def group(g, carry):
        for b in range(NBUF):
            pltpu.async_copy(ones_v, deg_sh.at[dst_v.at[g * NBUF + b]], sem, add=True)

        @pl.when(g > 0)
        def _():
            for b in range(NBUF):
                pltpu.make_async_copy(ones_hbm, ones_v, sem).wait()

        return carry

    lax.fori_loop(0, DGROUPS, group, 0)
    for b in range(NBUF):
        pltpu.make_async_copy(ones_hbm, ones_v, sem).wait()
    plsc.subcore_barrier()
    _per_tile_rows(s, lambda off, n: pltpu.sync_copy(
        deg_sh.at[pl.ds(off, n)], out_hbm.at[c, pl.ds(off, n)]))


@functools.partial(
    pl.kernel,
    out_type=jax.ShapeDtypeStruct((NC, N, DH), jnp.float32),
    mesh=_mesh,
    compiler_params=pltpu.CompilerParams(use_tc_tiling_on_sc=False),
    scratch_types=[
        pltpu.VMEM((CHUNKS, C), jnp.int32),
        pltpu.VMEM((CHUNKS, C), jnp.int32),
        pltpu.VMEM((NBUF, C, DH), jnp.float32),
        pltpu.VMEM_SHARED((N, DH), jnp.float32),
        pltpu.SemaphoreType.DMA((2,)),
    ],
)
def _edge_kernel(hd_hbm, src_hbm, dst_hbm, out_hbm,
                 src_v, dst_v, ring, acc_sh, sem):
    c = lax.axis_index("c")
    s = lax.axis_index("s")
    pltpu.sync_copy(src_hbm.at[s], src_v)
    pltpu.sync_copy(dst_hbm.at[s], dst_v)

    hd_c = hd_hbm.at[c]

    def fire_gathers(g):
        for j in range(NBUF):
            pltpu.async_copy(hd_c.at[src_v.at[g * NBUF + j]], ring.at[j],
                             sem.at[0])

    def drain_scatters():
        for j in range(NBUF):
            pltpu.make_async_copy(hd_c.at[pl.ds(0, C)], ring.at[j],
                                  sem.at[1]).wait()

    fire_gathers(0)
    _per_tile_rows(s, lambda off, n: pltpu.sync_copy(
        hd_hbm.at[c, pl.ds(off, n)], acc_sh.at[pl.ds(off, n)]))
    plsc.subcore_barrier()

    def group(g, carry):
        for b in range(NBUF):
            pltpu.make_async_copy(hd_c.at[pl.ds(0, C)], ring.at[b],
                                  sem.at[0]).wait()
            pltpu.async_copy(ring.at[b], acc_sh.at[dst_v.at[g * NBUF + b]],
                             sem.at[1], add=True)

        @pl.when(g + 1 < NGROUPS)
        def _():
            for b in range(NBUF):
                pltpu.make_async_copy(hd_c.at[pl.ds(0, C)], ring.at[b],
                                      sem.at[1]).wait()
                pltpu.async_copy(hd_c.at[src_v.at[(g + 1) * NBUF + b]],
                                 ring.at[b], sem.at[0])

        @pl.when(g + 1 == NGROUPS)
        def _():
            drain_scatters()

        return carry

    lax.fori_loop(0, NGROUPS, group, 0)
    plsc.subcore_barrier()
    _per_tile_rows(s, lambda off, n: pltpu.sync_copy(
        acc_sh.at[pl.ds(off, n)], out_hbm.at[c, pl.ds(off, n)]))


def _d_block(degp_ref):
    deg = 1.0 + degp_ref[0, :, 0:1] + degp_ref[1, :, 0:1]
    return lax.rsqrt(deg)


def _split_store(o_ref, res):
    o_ref[0, :, :] = res[:, :DH]
    o_ref[1, :, :] = res[:, DH:]


def _mm1_body(x_ref, w_ref, degp_ref, o_ref):
    d = _d_block(degp_ref)
    _split_store(o_ref, d * jnp.dot(x_ref[...], w_ref[...],
                                    preferred_element_type=jnp.float32))


def _mm2_body(q_ref, degp_ref, b1_ref, w2_ref, o_ref):
    d = _d_block(degp_ref)
    q = jnp.concatenate([q_ref[0], q_ref[1]], axis=-1)
    t = jnp.maximum(d * q + b1_ref[...], 0.0)
    _split_store(o_ref, d * jnp.dot(t, w2_ref[...],
                                    preferred_element_type=jnp.float32))


def _comb_body(r_ref, degp_ref, b2_ref, o_ref):
    d = _d_block(degp_ref)
    r = jnp.concatenate([r_ref[0], r_ref[1]], axis=-1)
    o_ref[...] = d * r + b2_ref[...]


_row_spec = pl.BlockSpec((BM, D), lambda i: (i, 0))
_half_spec = pl.BlockSpec((NC, BM, DH), lambda i: (0, i, 0))
_deg_spec = pl.BlockSpec((NC, BM, DW), lambda i: (0, i, 0))
_w_spec = pl.BlockSpec((D, D), lambda i: (0, 0))
_b_spec = pl.BlockSpec((1, D), lambda i: (0, 0))
_half_out = jax.ShapeDtypeStruct((NC, N, DH), jnp.float32)

_mm1 = pl.pallas_call(
    _mm1_body, grid=(N // BM,),
    in_specs=[_row_spec, _w_spec, _deg_spec],
    out_specs=_half_spec, out_shape=_half_out)

_mm2 = pl.pallas_call(
    _mm2_body, grid=(N // BM,),
    in_specs=[_half_spec, _deg_spec, _b_spec, _w_spec],
    out_specs=_half_spec, out_shape=_half_out)

_comb = pl.pallas_call(
    _comb_body, grid=(N // BM,),
    in_specs=[_half_spec, _deg_spec, _b_spec],
    out_specs=pl.BlockSpec((BM, D), lambda i: (i, 0)),
    out_shape=jax.ShapeDtypeStruct((N, D), jnp.float32))


@jax.jit
def kernel(x, edge_index, W1, b1, W2, b2):
    ei = edge_index.astype(jnp.int32)
    src = ei[0].reshape(NS, CHUNKS, C)
    dst = ei[1].reshape(NS, CHUNKS, C)
    dstd = ei[1].reshape(NC * NS, DCHUNKS, C)
    ones = jnp.ones((C, DW), jnp.float32)
    zdeg = jnp.zeros((N, DW), jnp.float32)
    b1r = b1.reshape(1, D)
    b2r = b2.reshape(1, D)

    degp = _deg_kernel(dstd, ones, zdeg)
    hd1 = _mm1(x, W1, degp)
    q = _edge_kernel(hd1, src, dst)
    hd2 = _mm2(q, degp, b1r, W2)
    r = _edge_kernel(hd2, src, dst)
    return _comb(r, degp, b2r)
